# baseline (speedup 1.0000x reference)
import jax
import jax.numpy as jnp
from jax.experimental import pallas as pl
from jax.experimental.pallas import tpu as pltpu

_P = 100000


def _body(g_ref, i_ref, x_ref, y_ref, out_ref):
    gx = g_ref[0, 0]
    gy = g_ref[0, 1]
    idx = i_ref[0, 0]
    m = (x_ref[...] == gx) & (y_ref[...] == gy)
    out_ref[...] = jnp.where(m, idx, 0)


def kernel(nodes):
    xs = nodes[:, 0]
    ys = nodes[:, 1]
    graph_nodes = jnp.array([[0, 0]], dtype=jnp.int32)
    indices = jnp.arange(graph_nodes.shape[0], dtype=jnp.int32)
    gbuf = graph_nodes.astype(jnp.float32)
    ibuf = indices.reshape(1, 1)
    return pl.pallas_call(
        _body,
        in_specs=[
            pl.BlockSpec(memory_space=pltpu.SMEM),
            pl.BlockSpec(memory_space=pltpu.SMEM),
            pl.BlockSpec((_P,), lambda: (0,)),
            pl.BlockSpec((_P,), lambda: (0,)),
        ],
        out_specs=pl.BlockSpec((_P,), lambda: (0,)),
        out_shape=jax.ShapeDtypeStruct((_P,), jnp.int32),
    )(gbuf, ibuf, xs, ys)


# speedup vs baseline: 2.8607x; 2.8607x over previous
import jax
import jax.numpy as jnp
from jax.experimental import pallas as pl

_P = 100000
_B = 16384


def _body(out_ref):
    out_ref[...] = jnp.zeros((_B,), jnp.int32)


def kernel(nodes):
    return pl.pallas_call(
        _body,
        grid=((_P + _B - 1) // _B,),
        out_specs=pl.BlockSpec((_B,), lambda i: (i,)),
        out_shape=jax.ShapeDtypeStruct((_P,), jnp.int32),
    )()


# speedup vs baseline: 8.9484x; 3.1281x over previous
import jax
import jax.numpy as jnp
from jax.experimental import pallas as pl

_P = 100000


def _masked_index_sum_body(out_ref):
    out_ref[...] = jnp.zeros((_P,), jnp.int32)


def kernel(nodes):
    original_shape = nodes.shape
    out = pl.pallas_call(
        _masked_index_sum_body,
        out_specs=pl.BlockSpec((_P,), lambda: (0,)),
        out_shape=jax.ShapeDtypeStruct((_P,), jnp.int32),
    )()
    return out.reshape(original_shape[:-1])
